# baseline (speedup 1.0000x reference)
import jax
import jax.numpy as jnp
from jax import lax
from jax.experimental import pallas as pl
from jax.experimental.pallas import tpu as pltpu
from jax.experimental.pallas import tpu_sc as plsc

N = 100000
DEG = 16
D = 128
STEPS = 16
K = 352
NW = 32
CPW = 9
VPW = K // 16


def _body(x_hbm, adj_hbm, ch_hbm, walks_hbm, acc_hbm, *scr):
    idx = (scr[0:STEPS + 1], scr[STEPS + 1:2 * STEPS + 2])
    ch = (scr[2 * STEPS + 2:3 * STEPS + 2],
          scr[3 * STEPS + 2:4 * STEPS + 2])
    eidx = scr[4 * STEPS + 2:4 * STEPS + 4]
    acc = scr[4 * STEPS + 4:4 * STEPS + 6]
    (sem_ch, sem_i0, sem_i1, sem_a0, sem_a1,
     sem_r0, sem_r1, sem_w) = scr[4 * STEPS + 6:]
    sem_i = (sem_i0, sem_i1)
    sem_a = (sem_a0, sem_a1)
    sem_r = (sem_r0, sem_r1)

    nc = plsc.get_sparse_core_info().num_cores
    wid = lax.axis_index("s") * nc + lax.axis_index("c")

    def base_of(t):
        return jnp.minimum((wid + NW * t) * K, N - K)

    def fire_ch(t, side):
        base = base_of(t)
        for s in range(STEPS):
            pltpu.async_copy(ch_hbm.at[pl.ds(s * N + base, K)],
                             ch[side][s], sem_ch)

    def drain_ch(n):
        for _ in range(n):
            pltpu.make_async_copy(ch_hbm.at[pl.ds(0, K)], ch[0][0],
                                  sem_ch).wait()

    def seed(t, side):
        base = base_of(t)

        def init(j, _):
            idx[side][0][pl.ds(16 * j, 16)] = (lax.iota(jnp.int32, 16)
                                               + base + 16 * j)
            return 0
        lax.fori_loop(0, VPW, init, 0)
        pltpu.async_copy(idx[side][0], walks_hbm.at[pl.ds(base, K)], sem_w)
        return pltpu.async_copy(x_hbm.at[idx[side][0]], acc[side], sem_i[side])

    def eidx_compute(side, s):
        def f(j, _):
            cur = idx[side][s][pl.ds(16 * j, 16)]
            c = ch[side][s][pl.ds(16 * j, 16)]
            eidx[side][pl.ds(16 * j, 16)] = cur * DEG + (c & (DEG - 1))
            return 0
        lax.fori_loop(0, VPW, f, 0)

    def fire_tail(t, side, s):
        base = base_of(t)
        pltpu.async_copy(x_hbm.at[idx[side][s + 1]], acc[side],
                         sem_r[side], add=True)
        pltpu.async_copy(idx[side][s + 1],
                         walks_hbm.at[pl.ds((s + 1) * N + base, K)], sem_w)

    def drain_rows_write(t, side):
        base = base_of(t)
        for _ in range(STEPS):
            pltpu.make_async_copy(x_hbm.at[pl.ds(0, K)], acc[side],
                                  sem_r[side]).wait()
        pltpu.sync_copy(acc[side], acc_hbm.at[pl.ds(base, K)])

    def drain_walks(n):
        for _ in range(n):
            pltpu.make_async_copy(walks_hbm.at[pl.ds(0, K)], eidx[0],
                                  sem_w).wait()

    def pair(u, prefetch_next):
        tA, tB = 2 * u, 2 * u + 1
        drain_ch(2 * STEPS)
        dA = seed(tA, 0)
        dB = seed(tB, 1)
        for s in range(STEPS):
            eidx_compute(0, s)
            a0 = pltpu.async_copy(adj_hbm.at[eidx[0]], idx[0][s + 1], sem_a[0])
            eidx_compute(1, s)
            a1 = pltpu.async_copy(adj_hbm.at[eidx[1]], idx[1][s + 1], sem_a[1])
            a0.wait()
            if s == 0:
                dA.wait()
            fire_tail(tA, 0, s)
            a1.wait()
            if s == 0:
                dB.wait()
            fire_tail(tB, 1, s)
        if prefetch_next:
            fire_ch(2 * u + 2, 0)
            fire_ch(2 * u + 3, 1)
        drain_rows_write(tA, 0)
        drain_rows_write(tB, 1)
        drain_walks(2 * (STEPS + 1))

    fire_ch(0, 0)
    fire_ch(1, 1)

    def pipe(u, _):
        pair(u, True)
        return 0
    lax.fori_loop(0, CPW // 2, pipe, 0)

    t = CPW - 1
    drain_ch(STEPS)
    dA = seed(t, 0)
    for s in range(STEPS):
        eidx_compute(0, s)
        pltpu.async_copy(adj_hbm.at[eidx[0]], idx[0][s + 1], sem_a[0]).wait()
        if s == 0:
            dA.wait()
        fire_tail(t, 0, s)
    drain_rows_write(t, 0)
    drain_walks(STEPS + 1)
    drain_ch(STEPS)


@jax.jit
def _walker(x, adj_nodes, choices):
    mesh = plsc.VectorSubcoreMesh(core_axis_name="c", subcore_axis_name="s")
    run = pl.kernel(
        _body,
        out_type=(
            jax.ShapeDtypeStruct(((STEPS + 1) * N,), jnp.int32),
            jax.ShapeDtypeStruct((N, D), jnp.float32),
        ),
        mesh=mesh,
        scratch_types=(
            [pltpu.VMEM((K,), jnp.int32) for _ in range(2 * (STEPS + 1))]
            + [pltpu.VMEM((K,), jnp.int32) for _ in range(2 * STEPS)]
            + [pltpu.VMEM((K,), jnp.int32) for _ in range(2)]
            + [pltpu.VMEM((K, D), jnp.float32) for _ in range(2)]
            + [pltpu.SemaphoreType.DMA] * 8
        ),
    )
    walks_flat, acc = run(x, adj_nodes, choices.reshape(-1))
    return walks_flat.reshape(STEPS + 1, N), acc


def kernel(x, adj_nodes, adj_offset, degrees, choices):
    del adj_offset, degrees
    return _walker(x, adj_nodes, choices)


# speedup vs baseline: 1.0101x; 1.0101x over previous
import jax
import jax.numpy as jnp
from jax import lax
from jax.experimental import pallas as pl
from jax.experimental.pallas import tpu as pltpu
from jax.experimental.pallas import tpu_sc as plsc

N = 100000
DEG = 16
D = 128
STEPS = 16
K = 640
NW = 32
CPW = 5
VPW = K // 16


def _body(x_hbm, adj_hbm, ch_hbm, walks_hbm, acc_hbm, *scr):
    idx_r = scr[0:STEPS + 1]
    ch_r = scr[STEPS + 1:2 * STEPS + 1]
    eidx_v = scr[2 * STEPS + 1]
    acc_v = scr[2 * STEPS + 2]
    sem_ch, sem_init, sem_adj, sem_rows, sem_w = scr[2 * STEPS + 3:]

    nc = plsc.get_sparse_core_info().num_cores
    wid = lax.axis_index("s") * nc + lax.axis_index("c")

    def chunk(t, _):
        base = jnp.minimum((wid + NW * t) * K, N - K)

        ch_d = [pltpu.async_copy(ch_hbm.at[pl.ds(s * N + base, K)],
                                 ch_r[s], sem_ch)
                for s in range(STEPS)]

        def init(j, _):
            idx_r[0][pl.ds(16 * j, 16)] = lax.iota(jnp.int32, 16) + base + 16 * j
            return 0
        lax.fori_loop(0, VPW, init, 0)

        w_d = [pltpu.async_copy(idx_r[0], walks_hbm.at[pl.ds(base, K)],
                                sem_w)]
        init_d = pltpu.async_copy(x_hbm.at[idx_r[0]], acc_v, sem_init)

        row_d = []
        for s in range(STEPS):
            ch_d[s].wait()

            def eidx(j, _):
                cur = idx_r[s][pl.ds(16 * j, 16)]
                c = ch_r[s][pl.ds(16 * j, 16)]
                eidx_v[pl.ds(16 * j, 16)] = cur * DEG + (c & (DEG - 1))
                return 0
            lax.fori_loop(0, VPW, eidx, 0)

            pltpu.async_copy(adj_hbm.at[eidx_v], idx_r[s + 1],
                             sem_adj).wait()
            w_d.append(pltpu.async_copy(
                idx_r[s + 1],
                walks_hbm.at[pl.ds((s + 1) * N + base, K)], sem_w))
            if s == 0:
                init_d.wait()
            row_d.append(pltpu.async_copy(x_hbm.at[idx_r[s + 1]],
                                          acc_v, sem_rows, add=True))

        for d in row_d:
            d.wait()
        for d in w_d:
            d.wait()
        pltpu.sync_copy(acc_v, acc_hbm.at[pl.ds(base, K)])
        return 0

    lax.fori_loop(0, CPW, chunk, 0)


@jax.jit
def _walker(x, adj_nodes, choices):
    mesh = plsc.VectorSubcoreMesh(core_axis_name="c", subcore_axis_name="s")
    run = pl.kernel(
        _body,
        out_type=(
            jax.ShapeDtypeStruct(((STEPS + 1) * N,), jnp.int32),
            jax.ShapeDtypeStruct((N, D), jnp.float32),
        ),
        mesh=mesh,
        scratch_types=(
            [pltpu.VMEM((K,), jnp.int32) for _ in range(STEPS + 1)]
            + [pltpu.VMEM((K,), jnp.int32) for _ in range(STEPS)]
            + [pltpu.VMEM((K,), jnp.int32),
               pltpu.VMEM((K, D), jnp.float32)]
            + [pltpu.SemaphoreType.DMA] * 5
        ),
    )
    walks_flat, acc = run(x, adj_nodes, choices.reshape(-1))
    return walks_flat.reshape(STEPS + 1, N), acc


def kernel(x, adj_nodes, adj_offset, degrees, choices):
    del adj_offset, degrees
    return _walker(x, adj_nodes, choices)


# speedup vs baseline: 1.0350x; 1.0247x over previous
import jax
import jax.numpy as jnp
from jax import lax
from jax.experimental import pallas as pl
from jax.experimental.pallas import tpu as pltpu
from jax.experimental.pallas import tpu_sc as plsc

N = 100000
DEG = 16
D = 128
STEPS = 16
K = 448
NW = 32
CPW = 7
VPW = K // 16


def _body(x_hbm, adj_hbm, ch_hbm, walks_hbm, acc_hbm, *scr):
    idx_r = scr[0:STEPS + 1]
    ch_r = scr[STEPS + 1:2 * STEPS + 1]
    eidx_v = scr[2 * STEPS + 1]
    acc_v = scr[2 * STEPS + 2]
    sem_ch, sem_init, sem_adj, sem_rows, sem_w = scr[2 * STEPS + 3:]

    nc = plsc.get_sparse_core_info().num_cores
    wid = lax.axis_index("s") * nc + lax.axis_index("c")

    def chunk(t, _):
        base = jnp.minimum((wid + NW * t) * K, N - K)

        ch_d = [pltpu.async_copy(ch_hbm.at[pl.ds(s * N + base, K)],
                                 ch_r[s], sem_ch)
                for s in range(STEPS)]

        def init(j, _):
            idx_r[0][pl.ds(16 * j, 16)] = lax.iota(jnp.int32, 16) + base + 16 * j
            return 0
        lax.fori_loop(0, VPW, init, 0)

        w_d = [pltpu.async_copy(idx_r[0], walks_hbm.at[pl.ds(base, K)],
                                sem_w)]
        init_d = pltpu.async_copy(x_hbm.at[idx_r[0]], acc_v, sem_init)

        row_d = []
        for s in range(STEPS):
            ch_d[s].wait()

            def eidx(j, _):
                cur = idx_r[s][pl.ds(16 * j, 16)]
                c = ch_r[s][pl.ds(16 * j, 16)]
                eidx_v[pl.ds(16 * j, 16)] = cur * DEG + (c & (DEG - 1))
                return 0
            lax.fori_loop(0, VPW, eidx, 0)

            pltpu.async_copy(adj_hbm.at[eidx_v], idx_r[s + 1],
                             sem_adj).wait()
            w_d.append(pltpu.async_copy(
                idx_r[s + 1],
                walks_hbm.at[pl.ds((s + 1) * N + base, K)], sem_w))
            if s == 0:
                init_d.wait()
            row_d.append(pltpu.async_copy(x_hbm.at[idx_r[s + 1]],
                                          acc_v, sem_rows, add=True))

        for d in row_d:
            d.wait()
        for d in w_d:
            d.wait()
        pltpu.sync_copy(acc_v, acc_hbm.at[pl.ds(base, K)])
        return 0

    lax.fori_loop(0, CPW, chunk, 0)


@jax.jit
def _walker(x, adj_nodes, choices):
    mesh = plsc.VectorSubcoreMesh(core_axis_name="c", subcore_axis_name="s")
    run = pl.kernel(
        _body,
        out_type=(
            jax.ShapeDtypeStruct(((STEPS + 1) * N,), jnp.int32),
            jax.ShapeDtypeStruct((N, D), jnp.float32),
        ),
        mesh=mesh,
        scratch_types=(
            [pltpu.VMEM((K,), jnp.int32) for _ in range(STEPS + 1)]
            + [pltpu.VMEM((K,), jnp.int32) for _ in range(STEPS)]
            + [pltpu.VMEM((K,), jnp.int32),
               pltpu.VMEM((K, D), jnp.float32)]
            + [pltpu.SemaphoreType.DMA] * 5
        ),
    )
    walks_flat, acc = run(x, adj_nodes, choices.reshape(-1))
    return walks_flat.reshape(STEPS + 1, N), acc


def kernel(x, adj_nodes, adj_offset, degrees, choices):
    del adj_offset, degrees
    return _walker(x, adj_nodes, choices)


# speedup vs baseline: 1.0357x; 1.0007x over previous
import jax
import jax.numpy as jnp
from jax import lax
from jax.experimental import pallas as pl
from jax.experimental.pallas import tpu as pltpu
from jax.experimental.pallas import tpu_sc as plsc

N = 100000
DEG = 16
D = 128
STEPS = 16
K = 448
NW = 32
CPW = 7
VPW = K // 16


def _body(x_hbm, adj_hbm, ch_hbm, walks_hbm, acc_hbm, *scr):
    idx_r = scr[0:STEPS + 1]
    ch_r = scr[STEPS + 1:2 * STEPS + 1]
    eidx_v = scr[2 * STEPS + 1]
    acc_v = scr[2 * STEPS + 2]
    sem_ch, sem_init, sem_adj, sem_rows, sem_w = scr[2 * STEPS + 3:]

    nc = plsc.get_sparse_core_info().num_cores
    wid = lax.axis_index("s") * nc + lax.axis_index("c")

    def chunk(t, _):
        base = jnp.minimum((wid + NW * t) * K, N - K)

        ch_d = [pltpu.async_copy(ch_hbm.at[pl.ds(s * N + base, K)],
                                 ch_r[s], sem_ch)
                for s in range(STEPS)]

        def init(j, _):
            idx_r[0][pl.ds(16 * j, 16)] = lax.iota(jnp.int32, 16) + base + 16 * j
            return 0
        lax.fori_loop(0, VPW, init, 0)

        w_d = [pltpu.async_copy(idx_r[0], walks_hbm.at[pl.ds(base, K)],
                                sem_w)]
        init_d = pltpu.async_copy(x_hbm.at[idx_r[0]], acc_v, sem_init)

        row_d = []
        for s in range(STEPS):
            ch_d[s].wait()

            def eidx(j, _):
                cur = idx_r[s][pl.ds(16 * j, 16)]
                c = ch_r[s][pl.ds(16 * j, 16)]
                eidx_v[pl.ds(16 * j, 16)] = cur * DEG + (c & (DEG - 1))
                return 0
            lax.fori_loop(0, VPW, eidx, 0)

            pltpu.async_copy(adj_hbm.at[eidx_v], idx_r[s + 1],
                             sem_adj, priority=1).wait()
            w_d.append(pltpu.async_copy(
                idx_r[s + 1],
                walks_hbm.at[pl.ds((s + 1) * N + base, K)], sem_w))
            if s == 0:
                init_d.wait()
            row_d.append(pltpu.async_copy(x_hbm.at[idx_r[s + 1]],
                                          acc_v, sem_rows, add=True))

        for d in row_d:
            d.wait()
        for d in w_d:
            d.wait()
        pltpu.sync_copy(acc_v, acc_hbm.at[pl.ds(base, K)])
        return 0

    lax.fori_loop(0, CPW, chunk, 0)


@jax.jit
def _walker(x, adj_nodes, choices):
    mesh = plsc.VectorSubcoreMesh(core_axis_name="c", subcore_axis_name="s")
    run = pl.kernel(
        _body,
        out_type=(
            jax.ShapeDtypeStruct(((STEPS + 1) * N,), jnp.int32),
            jax.ShapeDtypeStruct((N, D), jnp.float32),
        ),
        mesh=mesh,
        scratch_types=(
            [pltpu.VMEM((K,), jnp.int32) for _ in range(STEPS + 1)]
            + [pltpu.VMEM((K,), jnp.int32) for _ in range(STEPS)]
            + [pltpu.VMEM((K,), jnp.int32),
               pltpu.VMEM((K, D), jnp.float32)]
            + [pltpu.SemaphoreType.DMA] * 5
        ),
    )
    walks_flat, acc = run(x, adj_nodes, choices.reshape(-1))
    return walks_flat.reshape(STEPS + 1, N), acc


def kernel(x, adj_nodes, adj_offset, degrees, choices):
    del adj_offset, degrees
    return _walker(x, adj_nodes, choices)
